# baseline (speedup 1.0000x reference)
import functools

import jax
import jax.numpy as jnp
from jax import lax
from jax.experimental import pallas as pl
from jax.experimental.pallas import tpu as pltpu
from jax.experimental.pallas import tpu_sc as plsc

_NC, _NS, _LANES = 2, 16, 16
_NW = _NC * _NS
_CHUNK = 16
_NB = 5
_VSTEP = 8


def _sc_body(B, L, D, x_hbm, idx_hbm, table_hbm, out_hbm,
             idx_v, tbuf, xbuf, *sems):
    tsems = sems[0:2]
    isems = sems[2:2 + _NB]
    osems = sems[2 + _NB:2 + 2 * _NB]
    rows_w = L // _NW
    nch = rows_w // _CHUNK
    T = nch * B
    wid = lax.axis_index("s") * _NC + lax.axis_index("c")
    base = wid * rows_w
    nvec = D // _LANES
    sh = nvec.bit_length() - 1

    pltpu.sync_copy(idx_hbm.at[pl.ds(wid * nch, nch), :], idx_v)

    tdesc, idesc, odesc = {}, {}, {}

    def start_chunk(ci):
        tdesc[ci] = pltpu.async_copy(table_hbm.at[idx_v.at[ci]],
                                     tbuf.at[ci & 1], tsems[ci & 1])

    def start_in(t):
        ci, b = divmod(t, B)
        s = t % _NB
        if t - _NB >= 0:
            odesc.pop(t - _NB).wait()
        idesc[t] = pltpu.async_copy(
            x_hbm.at[b, pl.ds(base + ci * _CHUNK, _CHUNK), :], xbuf.at[s],
            isems[s])

    start_chunk(0)
    start_chunk(1)
    start_in(0)
    start_in(1)
    start_in(2)

    for t in range(T):
        ci, b = divmod(t, B)
        s = t % _NB
        p = ci & 1
        if t + 3 < T:
            start_in(t + 3)
        if b == 0:
            tdesc.pop(ci).wait()
        idesc.pop(t).wait()

        @plsc.parallel_loop(0, _CHUNK * nvec, step=_VSTEP)
        def _add(v):
            r = v >> sh
            k0 = (v & (nvec - 1)) << 4
            for j in range(_VSTEP):
                k = pl.multiple_of(k0 + j * _LANES, _LANES)
                plsc.addupdate(xbuf.at[s, r, pl.ds(k, _LANES)],
                               tbuf[p, r, pl.ds(k, _LANES)])

        odesc[t] = pltpu.async_copy(
            xbuf.at[s], out_hbm.at[b, pl.ds(base + ci * _CHUNK, _CHUNK), :],
            osems[s])
        if b == B - 1 and ci + 2 < nch:
            start_chunk(ci + 2)

    for t in sorted(odesc):
        odesc.pop(t).wait()


def kernel(x, start, table):
    B, L, D = x.shape
    idx = (jnp.arange(L, dtype=jnp.int32)
           + jnp.asarray(start, jnp.int32)).reshape(L // _CHUNK, _CHUNK)

    mesh = plsc.VectorSubcoreMesh(core_axis_name="c", subcore_axis_name="s")
    sc = pl.kernel(
        functools.partial(_sc_body, B, L, D),
        out_type=jax.ShapeDtypeStruct((B, L, D), x.dtype),
        mesh=mesh,
        scratch_types=[
            pltpu.VMEM((L // _NW // _CHUNK, _CHUNK), jnp.int32),
            pltpu.VMEM((2, _CHUNK, D), jnp.float32),
            pltpu.VMEM((_NB, _CHUNK, D), jnp.float32),
        ] + [pltpu.SemaphoreType.DMA] * (2 + 2 * _NB),
    )
    return sc(x, idx, table)


# speedup vs baseline: 1.0077x; 1.0077x over previous
import functools

import jax
import jax.numpy as jnp
from jax import lax
from jax.experimental import pallas as pl
from jax.experimental.pallas import tpu as pltpu
from jax.experimental.pallas import tpu_sc as plsc

_NC, _NS, _LANES = 2, 16, 16
_NW = _NC * _NS
_CHUNK = 16
_NB = 4
_VSTEP = 8


def _sc_body(B, L, D, x_hbm, idx_hbm, table_hbm, out_hbm,
             idx_v, tbuf, xbuf, *sems):
    tsems = sems[0:3]
    isems = sems[3:3 + _NB]
    osems = sems[3 + _NB:3 + 2 * _NB]
    rows_w = L // _NW
    nch = rows_w // _CHUNK
    T = nch * B
    wid = lax.axis_index("s") * _NC + lax.axis_index("c")
    base = wid * rows_w
    nvec = D // _LANES
    sh = nvec.bit_length() - 1

    pltpu.sync_copy(idx_hbm.at[pl.ds(wid * nch, nch), :], idx_v)

    tdesc, idesc, odesc = {}, {}, {}

    def start_chunk(ci):
        tdesc[ci] = pltpu.async_copy(table_hbm.at[idx_v.at[ci]],
                                     tbuf.at[ci % 3], tsems[ci % 3])

    def start_in(t):
        ci, b = divmod(t, B)
        s = t % _NB
        if t - _NB >= 0:
            odesc.pop(t - _NB).wait()
        idesc[t] = pltpu.async_copy(
            x_hbm.at[b, pl.ds(base + ci * _CHUNK, _CHUNK), :], xbuf.at[s],
            isems[s])

    start_chunk(0)
    start_chunk(1)
    start_in(0)
    start_in(1)

    for t in range(T):
        ci, b = divmod(t, B)
        s = t % _NB
        p = ci % 3
        if t + 2 < T:
            start_in(t + 2)
        if b == 0:
            if ci + 2 < nch:
                start_chunk(ci + 2)
            tdesc.pop(ci).wait()
        idesc.pop(t).wait()

        @plsc.parallel_loop(0, _CHUNK * nvec, step=_VSTEP)
        def _add(v):
            r = v >> sh
            k0 = (v & (nvec - 1)) << 4
            for j in range(_VSTEP):
                k = pl.multiple_of(k0 + j * _LANES, _LANES)
                plsc.addupdate(xbuf.at[s, r, pl.ds(k, _LANES)],
                               tbuf[p, r, pl.ds(k, _LANES)])

        odesc[t] = pltpu.async_copy(
            xbuf.at[s], out_hbm.at[b, pl.ds(base + ci * _CHUNK, _CHUNK), :],
            osems[s])

    for t in sorted(odesc):
        odesc.pop(t).wait()


def kernel(x, start, table):
    B, L, D = x.shape
    idx = (jnp.arange(L, dtype=jnp.int32)
           + jnp.asarray(start, jnp.int32)).reshape(L // _CHUNK, _CHUNK)

    mesh = plsc.VectorSubcoreMesh(core_axis_name="c", subcore_axis_name="s")
    sc = pl.kernel(
        functools.partial(_sc_body, B, L, D),
        out_type=jax.ShapeDtypeStruct((B, L, D), x.dtype),
        mesh=mesh,
        scratch_types=[
            pltpu.VMEM((L // _NW // _CHUNK, _CHUNK), jnp.int32),
            pltpu.VMEM((3, _CHUNK, D), jnp.float32),
            pltpu.VMEM((_NB, _CHUNK, D), jnp.float32),
        ] + [pltpu.SemaphoreType.DMA] * (3 + 2 * _NB),
    )
    return sc(x, idx, table)
